# baseline (speedup 1.0000x reference)
import jax
import jax.numpy as jnp
from jax.experimental import pallas as pl
from jax.experimental.pallas import tpu as pltpu

_B = 128
_D = 32
_L = 512
_PBLK = 128
_NSTEP = 12
_CH = 32
_ROWS = _PBLK * _D
_OFF = (29, 34, 38)
_TBL = 64
_H1 = 512
_H2 = 256


def _mlp_body(idxl_ref, tail_ref, tbl_ref, w1_ref, w1t_ref, b1_ref, w2_ref,
              b2_ref, w3_ref, b3_ref, out_ref, acc_ref, xt_ref):
    j = pl.program_id(0)

    @pl.when(j == 0)
    def _():
        acc_ref[...] = jnp.zeros_like(acc_ref)

    tblt = tbl_ref[...].astype(jnp.bfloat16)

    def gather(off, width):
        def _g():
            tw = tblt[:, off:off + width]
            for c in range(_PBLK // _CH):
                idf = idxl_ref[pl.ds(j, 1), c * _CH * _B:(c + 1) * _CH * _B]
                ids = jax.lax.broadcasted_iota(
                    jnp.int32, (width, _CH * _B), 0) + off
                oh = (ids == idf).astype(jnp.bfloat16)
                et = jax.lax.dot_general(
                    tw, oh, (((1,), (0,)), ((), ())),
                    preferred_element_type=jnp.float32
                ).astype(jnp.bfloat16)
                for g in range(_CH):
                    p = c * _CH + g
                    xt_ref[p * _D:(p + 1) * _D, :] = et[:, g * _B:(g + 1) * _B]
        return _g

    s1 = 512 // _PBLK
    s2 = 1024 // _PBLK
    pl.when(j == 0)(gather(0, 40))
    pl.when(jnp.logical_and(j >= 1, j < s1))(gather(29, 8))
    pl.when(j == s1)(gather(29, 16))
    pl.when(jnp.logical_and(j > s1, j <= s2))(gather(34, 8))
    pl.when(j > s2)(gather(38, 8))

    w1 = w1_ref[...].astype(jnp.bfloat16)
    acc_ref[...] += jax.lax.dot_general(
        xt_ref[...], w1, (((0,), (0,)), ((), ())),
        preferred_element_type=jnp.float32)

    @pl.when(j == _NSTEP - 1)
    def _():
        ohtail = (jax.lax.broadcasted_iota(jnp.int32, (_TBL, _B), 0)
                  == tail_ref[...]).astype(jnp.bfloat16)
        ettail = jax.lax.dot_general(
            tblt, ohtail, (((1,), (0,)), ((), ())),
            preferred_element_type=jnp.float32).astype(jnp.bfloat16)
        tail = jax.lax.dot_general(
            ettail, w1t_ref[...].astype(jnp.bfloat16),
            (((0,), (0,)), ((), ())), preferred_element_type=jnp.float32)
        h1 = jnp.maximum(acc_ref[...] + tail + b1_ref[...], 0.0)
        h2 = jax.lax.dot_general(
            h1.astype(jnp.bfloat16), w2_ref[...].astype(jnp.bfloat16),
            (((1,), (0,)), ((), ())), preferred_element_type=jnp.float32)
        h2 = jnp.maximum(h2 + b2_ref[...], 0.0)
        h3 = jax.lax.dot_general(
            h2.astype(jnp.bfloat16), w3_ref[...].astype(jnp.bfloat16),
            (((1,), (0,)), ((), ())), preferred_element_type=jnp.float32)
        out_ref[...] = h3 + b3_ref[...]


def kernel(rna_data, tissue_id, tissue_table, seq_table, sec_table,
           loop_table, W1, b1, W2, b2, W3, b3):
    tbl = jnp.zeros((_TBL, _D), jnp.float32)
    tbl = tbl.at[0:29].set(tissue_table)
    tbl = tbl.at[29:34].set(seq_table)
    tbl = tbl.at[34:38].set(sec_table)
    tbl = tbl.at[38:46].set(loop_table)
    tblt = tbl.T

    gidx = jnp.concatenate(
        [tissue_id.reshape(1, _B)]
        + [jnp.transpose(rna_data[:, :, ch], (1, 0)) + _OFF[ch]
           for ch in range(3)], axis=0)
    idxl = gidx[:_NSTEP * _PBLK].reshape(_NSTEP, _PBLK * _B)
    tail_idx = gidx[_NSTEP * _PBLK:]

    out = pl.pallas_call(
        _mlp_body,
        grid=(_NSTEP,),
        in_specs=[
            pl.BlockSpec((_NSTEP, _PBLK * _B), lambda j: (0, 0)),
            pl.BlockSpec((1, _B), lambda j: (0, 0)),
            pl.BlockSpec((_D, _TBL), lambda j: (0, 0)),
            pl.BlockSpec((_ROWS, _H1), lambda j: (j, 0)),
            pl.BlockSpec((_D, _H1), lambda j: (_NSTEP * _PBLK, 0)),
            pl.BlockSpec((1, _H1), lambda j: (0, 0)),
            pl.BlockSpec((_H1, _H2), lambda j: (0, 0)),
            pl.BlockSpec((1, _H2), lambda j: (0, 0)),
            pl.BlockSpec((_H2, 1), lambda j: (0, 0)),
            pl.BlockSpec((1, 1), lambda j: (0, 0)),
        ],
        out_specs=pl.BlockSpec((_B, 1), lambda j: (0, 0)),
        out_shape=jax.ShapeDtypeStruct((_B, 1), jnp.float32),
        scratch_shapes=[
            pltpu.VMEM((_B, _H1), jnp.float32),
            pltpu.VMEM((_ROWS, _B), jnp.bfloat16),
        ],
    )(idxl, tail_idx, tblt, W1, W1,
      b1.reshape(1, _H1), W2, b2.reshape(1, _H2), W3, b3.reshape(1, 1))
    return out
